# baseline (speedup 1.0000x reference)
import functools

import jax
import jax.numpy as jnp
from jax import lax
from jax.experimental import pallas as pl
from jax.experimental.pallas import tpu as pltpu
from jax.experimental.pallas import tpu_sc as plsc

N = 10000
E = 320000
D = 512
NW = 32
EPW = E // NW
NB = 16
BKT = 640
NPAD = NB * BKT
FS = 128
G = 128
TRASH = BKT
AGR = BKT + 8
LCAP = 10240
RING = 256
PCHUNK = 2000
RCHUNK = 1024

_mesh = plsc.VectorSubcoreMesh(core_axis_name="c", subcore_axis_name="s")
_params = pltpu.CompilerParams(needs_layout_passes=False)


@functools.partial(
    pl.kernel, mesh=_mesh,
    out_type=[jax.ShapeDtypeStruct((NB * NW * LCAP,), jnp.int32),
              jax.ShapeDtypeStruct((NW * 16,), jnp.int32)],
    compiler_params=_params,
    scratch_types=[
        pltpu.VMEM((PCHUNK,), jnp.int32),
        pltpu.VMEM((PCHUNK,), jnp.int32),
        pltpu.VMEM((NB * RING,), jnp.int32),
        pltpu.VMEM((16,), jnp.int32),
    ])
def _partition(ei, pairs, counts, src_v, dst_v, ring_v, cnt_v):
    wid = lax.axis_index("s") * 2 + lax.axis_index("c")
    iota16 = lax.iota(jnp.int32, 16)
    trash_pack = iota16 * 1024 + TRASH + lax.bitwise_and(iota16, 7)
    ebase = wid * EPW

    def flush(b, flushed):
        src_off = pl.multiple_of(
            b * RING + lax.bitwise_and(flushed, RING - 1), G)
        dst_off = pl.multiple_of((b * NW + wid) * LCAP + flushed, G)
        pltpu.sync_copy(ring_v.at[pl.ds(src_off, G)],
                        pairs.at[pl.ds(dst_off, G)])

    def fvreg(vsrc, vdst, state):
        bkt = lax.shift_right_logical(vdst * 6554, 22)
        packed = vsrc * 1024 + vdst - bkt * BKT
        fills = list(state[:NB])
        flushes = list(state[NB:])
        for b in range(NB):
            mask = bkt == b
            pref = plsc.cumsum(mask.astype(jnp.int32))
            pos = b * RING + lax.bitwise_and(fills[b] + pref - 1, RING - 1)
            plsc.store_scatter(ring_v, [pos], packed, mask=mask)
            fills[b] = fills[b] + pref[15]
        return tuple(fills) + tuple(flushes)

    def chunk(c, state):
        off = pl.multiple_of(ebase + c * PCHUNK, 16)
        pltpu.sync_copy(ei.at[pl.ds(off, PCHUNK)], src_v)
        pltpu.sync_copy(ei.at[pl.ds(E + off, PCHUNK)], dst_v)

        def super5(t, state):
            for u in range(5):
                i = t * 5 + u
                vsrc = src_v[pl.ds(i * 16, 16)]
                vdst = dst_v[pl.ds(i * 16, 16)]
                state = fvreg(vsrc, vdst, state)
            fills = list(state[:NB])
            flushes = list(state[NB:])
            for b in range(NB):
                @pl.when(fills[b] - flushes[b] >= G)
                def _(b=b, flushed=flushes[b]):
                    flush(b, flushed)

                flushes[b] = jnp.where(fills[b] - flushes[b] >= G,
                                       flushes[b] + G, flushes[b])
            return tuple(fills) + tuple(flushes)

        return lax.fori_loop(0, PCHUNK // 80, super5, state)

    state = tuple([jnp.int32(0)] * (2 * NB))
    state = lax.fori_loop(0, EPW // PCHUNK, chunk, state)

    fills = list(state[:NB])
    flushes = list(state[NB:])
    cvec = jnp.zeros((16,), jnp.int32)
    for b in range(NB):
        for k in range(8):
            idx = b * RING + lax.bitwise_and(fills[b] + k * 16 + iota16,
                                             RING - 1)
            plsc.store_scatter(ring_v, [idx], trash_pack)
        fill_p = lax.bitwise_and(fills[b] + G - 1, jnp.int32(-G))

        @pl.when(fill_p - flushes[b] >= G)
        def _(b=b, flushed=flushes[b]):
            flush(b, flushed)

        @pl.when(fill_p - flushes[b] >= 2 * G)
        def _(b=b, flushed=flushes[b] + G):
            flush(b, flushed)

        cvec = jnp.where(iota16 == b, fill_p, cvec)
    cnt_v[...] = cvec
    pltpu.sync_copy(cnt_v, counts.at[pl.ds(pl.multiple_of(wid * 16, 16),
                                            16)])


def _make_segsum(with_deg):
    out_type = [jax.ShapeDtypeStruct((NPAD, D), jnp.float32)]
    if with_deg:
        out_type.append(jax.ShapeDtypeStruct((NPAD * 16,), jnp.float32))
    scratch = [
        pltpu.VMEM((RCHUNK,), jnp.int32),
        pltpu.VMEM((G,), jnp.int32),
        pltpu.VMEM((G,), jnp.int32),
        pltpu.VMEM((G, FS), jnp.float32),
        pltpu.VMEM((AGR, FS), jnp.float32),
        pltpu.VMEM((AGR * 16,), jnp.float32),
        pltpu.VMEM((NW * 16,), jnp.int32),
        pltpu.SemaphoreType.DMA,
    ]

    @functools.partial(pl.kernel, mesh=_mesh, out_type=out_type,
                       compiler_params=_params, scratch_types=scratch)
    def segsum(h4, pairs, counts, *refs):
        if with_deg:
            out, deg_out = refs[0], refs[1]
            pj_v, gidx_v, ld_v, rows_v, agg_v, deg_v, cnt_v, sem = refs[2:]
        else:
            out = refs[0]
            deg_out = None
            pj_v, gidx_v, ld_v, rows_v, agg_v, deg_v, cnt_v, sem = refs[1:]

        wid = lax.axis_index("s") * 2 + lax.axis_index("c")
        zero16 = jnp.zeros((16,), jnp.float32)
        ones16 = jnp.ones((16,), jnp.float32)
        iota16 = lax.iota(jnp.int32, 16)

        pltpu.sync_copy(counts, cnt_v)

        def process_block(bbase, q):
            def ubody(i, _):
                p = pj_v[pl.ds(bbase + i * 16, 16)]
                gidx_v[pl.ds(i * 16, 16)] = \
                    lax.shift_right_logical(p, 10) * 4 + q
                ld_v[pl.ds(i * 16, 16)] = lax.bitwise_and(p, 1023)
                return 0

            lax.fori_loop(0, G // 16, ubody, 0)
            pltpu.async_copy(h4.at[gidx_v], rows_v, sem).wait()

            def jbody(i, _):
                ldv = ld_v[pl.ds(i * 16, 16)]
                for l in range(16):
                    s = ldv[l]
                    j = i * 16 + l
                    for k in range(8):
                        plsc.addupdate(agg_v.at[s, pl.ds(k * 16, 16)],
                                       rows_v[j, pl.ds(k * 16, 16)])
                return 0

            lax.fori_loop(0, G // 16, jbody, 0)
            if with_deg:
                @pl.when(q == 0)
                def _():
                    def dbody(i, _):
                        ldv = ld_v[pl.ds(i * 16, 16)]
                        plsc.addupdate_scatter(deg_v, [ldv * 16 + iota16],
                                               ones16)
                        return 0

                    lax.fori_loop(0, G // 16, dbody, 0)

        def do_job(job):
            b = job >> 2
            q = lax.bitwise_and(job, 3)
            lo = b * BKT

            def zrow(r, _):
                for k in range(8):
                    agg_v[r, pl.ds(k * 16, 16)] = zero16
                return 0

            lax.fori_loop(0, AGR, zrow, 0)
            if with_deg:
                @pl.when(q == 0)
                def _():
                    def zdeg(r, _):
                        deg_v[pl.ds(r * 16, 16)] = zero16
                        return 0

                    lax.fori_loop(0, AGR, zdeg, 0)

            def wloop(w2, _):
                cidx = jnp.zeros((16,), jnp.int32) + (w2 * 16 + b)
                cnt = plsc.load_gather(cnt_v, [cidx])[0]
                lbase = (b * NW + w2) * LCAP

                def rchunk(t, _):
                    roff = pl.multiple_of(lbase + t * RCHUNK, RCHUNK)
                    pltpu.sync_copy(
                        pairs.at[pl.ds(roff, RCHUNK)], pj_v)
                    nbl = lax.min(cnt - t * RCHUNK,
                                  jnp.int32(RCHUNK)) >> 7

                    def pb(g, _):
                        process_block(g * G, q)
                        return 0

                    lax.fori_loop(0, nbl, pb, 0)
                    return 0

                lax.fori_loop(0, (cnt + RCHUNK - 1) >> 10, rchunk, 0)
                return 0

            lax.fori_loop(0, NW, wloop, 0)

            pltpu.sync_copy(agg_v.at[pl.ds(0, BKT), :],
                            out.at[pl.ds(lo, BKT), pl.ds(q * FS, FS)])
            if with_deg:
                @pl.when(q == 0)
                def _():
                    pltpu.sync_copy(
                        deg_v.at[pl.ds(0, BKT * 16)],
                        deg_out.at[pl.ds(pl.multiple_of(lo * 16, 128),
                                         BKT * 16)])

        for jj in range(2):
            do_job(wid + 32 * jj)

    return segsum


_segsum_deg = _make_segsum(True)
_segsum = _make_segsum(False)


def _mm_body(x_ref, w_ref, o_ref):
    o_ref[...] = jnp.dot(x_ref[...], w_ref[...],
                         preferred_element_type=jnp.float32)


def _mm_in(x, w):
    return pl.pallas_call(
        _mm_body,
        grid=(5,),
        in_specs=[
            pl.BlockSpec((2000, 128), lambda i: (i, 0)),
            pl.BlockSpec((128, D), lambda i: (0, 0)),
        ],
        out_specs=pl.BlockSpec((2000, D), lambda i: (i, 0)),
        out_shape=jax.ShapeDtypeStruct((N, D), jnp.float32),
    )(x, w)


def _layer_body(sum_ref, deg_ref, h_ref, w_ref, o_ref):
    deg = jnp.sum(deg_ref[...], axis=1, keepdims=True)
    mean = sum_ref[...] / jnp.maximum(deg, 1.0)
    o_ref[...] = jax.nn.relu(
        jnp.dot(mean, w_ref[...], preferred_element_type=jnp.float32)
    ) + h_ref[...]


def _layer(agg_sum, deg16, h, w):
    return pl.pallas_call(
        _layer_body,
        grid=(5,),
        in_specs=[
            pl.BlockSpec((2000, D), lambda i: (i, 0)),
            pl.BlockSpec((2000, 16), lambda i: (i, 0)),
            pl.BlockSpec((2000, D), lambda i: (i, 0)),
            pl.BlockSpec((D, D), lambda i: (0, 0)),
        ],
        out_specs=pl.BlockSpec((2000, D), lambda i: (i, 0)),
        out_shape=jax.ShapeDtypeStruct((N, D), jnp.float32),
    )(agg_sum, deg16, h, w)


def _head_body(h_ref, wc1_ref, bc1_ref, wc2_ref, bc2_ref, o_ref):
    z = jax.nn.relu(
        jnp.dot(h_ref[...], wc1_ref[...],
                preferred_element_type=jnp.float32) + bc1_ref[...]
    )
    logits = jnp.dot(z, wc2_ref[...],
                     preferred_element_type=jnp.float32) + bc2_ref[...]
    m = jnp.max(logits, axis=0, keepdims=True)
    e = jnp.exp(logits - m)
    o_ref[...] = e / jnp.sum(e, axis=0, keepdims=True)


def _head(h, wc1, bc1, wc2, bc2):
    return pl.pallas_call(
        _head_body,
        out_shape=jax.ShapeDtypeStruct((N, 8), jnp.float32),
    )(h, wc1, bc1, wc2, bc2)


def kernel(x, edge_index, W_in, W1, W2, Wc1, bc1, Wc2, bc2):
    h0 = _mm_in(x, W_in)
    ei_flat = edge_index.reshape(-1)
    pairs, counts = _partition(ei_flat)
    sum1, deg_flat = _segsum_deg(h0.reshape(4 * N, FS), pairs, counts)
    deg16 = deg_flat.reshape(NPAD, 16)
    h1 = _layer(sum1, deg16, h0, W1)
    (sum2,) = _segsum(h1.reshape(4 * N, FS), pairs, counts)
    h2 = _layer(sum2, deg16, h1, W2)
    wc2p = jnp.pad(Wc2, ((0, 0), (0, 3)))
    bc2p = jnp.pad(bc2, (0, 3))
    out8 = _head(h2, Wc1, bc1.reshape(1, -1), wc2p, bc2p.reshape(1, -1))
    return out8[:, :5]


# speedup vs baseline: 2.4327x; 2.4327x over previous
import functools

import jax
import jax.numpy as jnp
from jax import lax
from jax.experimental import pallas as pl
from jax.experimental.pallas import tpu as pltpu
from jax.experimental.pallas import tpu_sc as plsc

N = 10000
E = 320000
D = 512
NW = 32
EPW = E // NW
NB = 16
BKT = 640
NPAD = NB * BKT
FS = 128
G = 128
TRASH = BKT
AGR = BKT + 8
LCAP = 10240
RING = 256
PCHUNK = 2000
RCHUNK = 1024

_mesh = plsc.VectorSubcoreMesh(core_axis_name="c", subcore_axis_name="s")
_params = pltpu.CompilerParams(needs_layout_passes=False)


EPW16 = 20096
NGR_FULL = 157
NGR_LAST = 145


def _make_segsum(with_deg):
    out_type = [jax.ShapeDtypeStruct((NPAD, D), jnp.float32)]
    if with_deg:
        out_type.append(jax.ShapeDtypeStruct((NPAD,), jnp.float32))
    scratch = [
        pltpu.VMEM((1024,), jnp.int32),
        pltpu.VMEM((1024,), jnp.int32),
        pltpu.VMEM((G,), jnp.int32),
        pltpu.VMEM((G,), jnp.int32),
        pltpu.VMEM((G, FS), jnp.float32),
        pltpu.VMEM((G,), jnp.float32),
        pltpu.VMEM((64, FS), jnp.float32),
        pltpu.VMEM((FS,), jnp.float32),
        pltpu.VMEM_SHARED((NPAD, FS), jnp.float32),
        pltpu.VMEM_SHARED((NPAD,), jnp.float32),
        pltpu.SemaphoreType.DMA,
    ]

    @functools.partial(pl.kernel, mesh=_mesh, out_type=out_type,
                       compiler_params=_params, scratch_types=scratch)
    def segsum(h4, ei, *refs):
        if with_deg:
            out, deg_out = refs[0], refs[1]
            (sv, dv, gidx_v, didx_v, rows_v, ones_v, zero_v, zd_v,
             agg_sh, deg_sh, sem) = refs[2:]
        else:
            out = refs[0]
            deg_out = None
            (sv, dv, gidx_v, didx_v, rows_v, ones_v, zero_v, zd_v,
             agg_sh, deg_sh, sem) = refs[1:]

        c = lax.axis_index("c")
        sidx = lax.axis_index("s")
        zero16 = jnp.zeros((16,), jnp.float32)
        base = sidx * EPW16
        ngroups = jnp.where(sidx == 15, NGR_LAST, NGR_FULL)
        rlo = pl.multiple_of(sidx * BKT, BKT)

        def zv(r, _):
            for k in range(8):
                zero_v[r, pl.ds(k * 16, 16)] = zero16
            return 0

        lax.fori_loop(0, 64, zv, 0)
        for k in range(8):
            ones_v[pl.ds(k * 16, 16)] = zero16 + 1.0
            zd_v[pl.ds(k * 16, 16)] = zero16

        def process_group(goff, f, do_deg):
            def ub(i, _):
                vs = sv[pl.ds(goff + i * 16, 16)]
                gidx_v[pl.ds(i * 16, 16)] = vs * 4 + f
                didx_v[pl.ds(i * 16, 16)] = dv[pl.ds(goff + i * 16, 16)]
                return 0

            lax.fori_loop(0, G // 16, ub, 0)
            pltpu.async_copy(h4.at[gidx_v], rows_v, sem).wait()
            pltpu.sync_copy(rows_v, agg_sh.at[didx_v], add=True)
            if with_deg:
                @pl.when(do_deg)
                def _():
                    pltpu.sync_copy(ones_v, deg_sh.at[didx_v], add=True)

        for p in range(2):
            f = 2 * p + c
            do_deg = jnp.logical_and(c == 0, p == 0)
            for t in range(BKT // 64):
                pltpu.sync_copy(zero_v,
                                agg_sh.at[pl.ds(rlo + t * 64, 64), :])
            if with_deg and p == 0:
                @pl.when(c == 0)
                def _():
                    for t in range(BKT // FS):
                        pltpu.sync_copy(
                            zd_v, deg_sh.at[pl.ds(rlo + t * FS, FS)])
            plsc.subcore_barrier()

            def chunk(t, _):
                goff = pl.multiple_of(base + t * 1024, 1024)
                pltpu.sync_copy(ei.at[pl.ds(goff, 1024)], sv)
                pltpu.sync_copy(ei.at[pl.ds(E + goff, 1024)], dv)
                for u in range(8):
                    @pl.when(t * 8 + u < ngroups)
                    def _(u=u):
                        process_group(u * G, f, do_deg)
                return 0

            lax.fori_loop(0, (NGR_FULL + 7) // 8, chunk, 0)
            plsc.subcore_barrier()
            pltpu.sync_copy(agg_sh.at[pl.ds(rlo, BKT), :],
                            out.at[pl.ds(rlo, BKT),
                                   pl.ds(pl.multiple_of(f * FS, FS), FS)])
            if with_deg and p == 0:
                @pl.when(c == 0)
                def _():
                    pltpu.sync_copy(deg_sh.at[pl.ds(rlo, BKT)],
                                    deg_out.at[pl.ds(rlo, BKT)])
            plsc.subcore_barrier()

    return segsum


_segsum_deg = _make_segsum(True)
_segsum = _make_segsum(False)


def _mm_body(x_ref, w_ref, o_ref):
    o_ref[...] = jnp.dot(x_ref[...], w_ref[...],
                         preferred_element_type=jnp.float32)


def _mm_in(x, w):
    return pl.pallas_call(
        _mm_body,
        grid=(5,),
        in_specs=[
            pl.BlockSpec((2000, 128), lambda i: (i, 0)),
            pl.BlockSpec((128, D), lambda i: (0, 0)),
        ],
        out_specs=pl.BlockSpec((2000, D), lambda i: (i, 0)),
        out_shape=jax.ShapeDtypeStruct((N, D), jnp.float32),
    )(x, w)


def _layer_body(sum_ref, deg_ref, h_ref, w_ref, o_ref):
    mean = sum_ref[...] / jnp.maximum(deg_ref[...], 1.0)
    o_ref[...] = jax.nn.relu(
        jnp.dot(mean, w_ref[...], preferred_element_type=jnp.float32)
    ) + h_ref[...]


def _layer(agg_sum, deg16, h, w):
    return pl.pallas_call(
        _layer_body,
        grid=(5,),
        in_specs=[
            pl.BlockSpec((2000, D), lambda i: (i, 0)),
            pl.BlockSpec((2000, 1), lambda i: (i, 0)),
            pl.BlockSpec((2000, D), lambda i: (i, 0)),
            pl.BlockSpec((D, D), lambda i: (0, 0)),
        ],
        out_specs=pl.BlockSpec((2000, D), lambda i: (i, 0)),
        out_shape=jax.ShapeDtypeStruct((N, D), jnp.float32),
    )(agg_sum, deg16, h, w)


def _head_body(h_ref, wc1_ref, bc1_ref, wc2_ref, bc2_ref, o_ref):
    z = jax.nn.relu(
        jnp.dot(h_ref[...], wc1_ref[...],
                preferred_element_type=jnp.float32) + bc1_ref[...]
    )
    logits = jnp.dot(z, wc2_ref[...],
                     preferred_element_type=jnp.float32) + bc2_ref[...]
    m = jnp.max(logits, axis=0, keepdims=True)
    e = jnp.exp(logits - m)
    o_ref[...] = e / jnp.sum(e, axis=0, keepdims=True)


def _head(h, wc1, bc1, wc2, bc2):
    return pl.pallas_call(
        _head_body,
        out_shape=jax.ShapeDtypeStruct((N, 8), jnp.float32),
    )(h, wc1, bc1, wc2, bc2)


def kernel(x, edge_index, W_in, W1, W2, Wc1, bc1, Wc2, bc2):
    h0 = _mm_in(x, W_in)
    ei_flat = jnp.concatenate(
        [edge_index.reshape(-1), jnp.zeros((2048,), jnp.int32)])
    sum1, deg = _segsum_deg(h0.reshape(4 * N, FS), ei_flat)
    deg1 = deg.reshape(NPAD, 1)
    h1 = _layer(sum1, deg1, h0, W1)
    (sum2,) = _segsum(h1.reshape(4 * N, FS), ei_flat)
    h2 = _layer(sum2, deg1, h1, W2)
    wc2p = jnp.pad(Wc2, ((0, 0), (0, 3)))
    bc2p = jnp.pad(bc2, (0, 3))
    out8 = _head(h2, Wc1, bc1.reshape(1, -1), wc2p, bc2p.reshape(1, -1))
    return out8[:, :5]
